# baseline (speedup 1.0000x reference)
import functools

import jax
import jax.numpy as jnp
import numpy as np
from jax.experimental import pallas as pl
from jax.experimental.pallas import tpu as pltpu

D_MODEL = 768
NUM_HEADS = 12
HEAD_DIM = 64
BS = 128
TOP_K = 3
CHUNK = 512
BPC = CHUNK // BS
HPG = 4
NGROUPS = NUM_HEADS // HPG

NEG_INF = float("-inf")


def _head_attention(q, k, v, seq_len):
    nb = seq_len // BS
    scale = 1.0 / np.sqrt(HEAD_DIM)

    k_mean = jnp.mean(k.reshape(nb, BS, HEAD_DIM), axis=1)
    gate = jax.lax.dot_general(
        k_mean, q, (((1,), (1,)), ((), ())),
        preferred_element_type=jnp.float32)
    blk = jax.lax.broadcasted_iota(jnp.int32, (nb, seq_len), 0)
    qblk = jax.lax.broadcasted_iota(jnp.int32, (nb, seq_len), 1) // BS
    gate = jnp.where(blk > qblk, NEG_INF, gate)

    sel = jnp.zeros((nb, seq_len), jnp.float32)
    g = gate
    for _ in range(TOP_K):
        m = jnp.max(g, axis=0, keepdims=True)
        is_max = g == m
        first_idx = jnp.min(jnp.where(is_max, blk, nb), axis=0,
                            keepdims=True)
        pick = blk == first_idx
        sel = jnp.maximum(sel, pick.astype(jnp.float32))
        g = jnp.where(pick, NEG_INF, g)
    w_t = sel * (blk < qblk).astype(jnp.float32)
    w = jnp.transpose(w_t)

    q16 = q.astype(jnp.bfloat16)
    k16 = k.astype(jnp.bfloat16)
    v16 = v.astype(jnp.bfloat16)

    r = jax.lax.broadcasted_iota(jnp.int32, (BS, BS), 0)
    c = jax.lax.broadcasted_iota(jnp.int32, (BS, BS), 1)
    causal_f = (c <= r).astype(jnp.float32)
    self_outs = []
    for i in range(nb):
        q_i = q16[i * BS:(i + 1) * BS, :]
        k_i = k16[i * BS:(i + 1) * BS, :]
        v_i = v16[i * BS:(i + 1) * BS, :]
        s_self = jax.lax.dot_general(
            q_i, k_i, (((1,), (1,)), ((), ())),
            preferred_element_type=jnp.float32) * scale
        e = jnp.exp(s_self) * causal_f
        den = jnp.sum(e, axis=1, keepdims=True)
        num = jax.lax.dot_general(
            e.astype(jnp.bfloat16), v_i, (((1,), (0,)), ((), ())),
            preferred_element_type=jnp.float32)
        self_outs.append(num / den)

    dr = jax.lax.broadcasted_iota(jnp.int32, (CHUNK, BPC), 0)
    dc = jax.lax.broadcasted_iota(jnp.int32, (CHUNK, BPC), 1)
    dmat = (dr // BS == dc).astype(jnp.bfloat16)

    adds = []
    for cidx in range(seq_len // CHUNK):
        row0 = (cidx * BPC + 1) * BS
        nrows = seq_len - row0
        q_c = q16[row0:, :]
        k_c = k16[cidx * CHUNK:(cidx + 1) * CHUNK, :]
        s = jax.lax.dot_general(
            q_c, k_c, (((1,), (1,)), ((), ())),
            preferred_element_type=jnp.float32) * scale
        e16 = jnp.exp(s).astype(jnp.bfloat16)
        den = jax.lax.dot_general(
            e16, dmat, (((1,), (0,)), ((), ())),
            preferred_element_type=jnp.float32)
        acc = None
        for b in range(BPC):
            blkidx = cidx * BPC + b
            v_b = v16[blkidx * BS:(blkidx + 1) * BS, :]
            num = jax.lax.dot_general(
                e16[:, b * BS:(b + 1) * BS], v_b, (((1,), (0,)), ((), ())),
                preferred_element_type=jnp.float32)
            coef = w[row0:, blkidx:blkidx + 1] / den[:, b:b + 1]
            contrib = num * coef
            acc = contrib if acc is None else acc + contrib
        adds.append((row0, acc))
    for row0, acc in adds:
        for i in range(row0 // BS, nb):
            self_outs[i] = self_outs[i] + acc[i * BS - row0:(i + 1) * BS - row0, :]
    return jnp.concatenate(self_outs, axis=0)


def _fused_body(x_ref, wq_ref, bq_ref, wk_ref, bk_ref, wv_ref, bv_ref,
                wo_ref, bo_ref, o_ref, scr_ref, *, seq_len):
    g = pl.program_id(0)
    xv = x_ref[:]
    dn = (((1,), (1,)), ((), ()))
    qg = jax.lax.dot_general(
        xv, wq_ref[:], dn, preferred_element_type=jnp.float32) + bq_ref[:]
    kg = jax.lax.dot_general(
        xv, wk_ref[:], dn, preferred_element_type=jnp.float32) + bk_ref[:]
    vg = jax.lax.dot_general(
        xv.astype(jnp.bfloat16), wv_ref[:].astype(jnp.bfloat16), dn,
        preferred_element_type=jnp.float32) + bv_ref[:]

    outs = []
    for hl in range(HPG):
        q = qg[:, hl * HEAD_DIM:(hl + 1) * HEAD_DIM]
        k = kg[:, hl * HEAD_DIM:(hl + 1) * HEAD_DIM]
        v = vg[:, hl * HEAD_DIM:(hl + 1) * HEAD_DIM]
        outs.append(_head_attention(q, k, v, seq_len))
    attn_g = jnp.concatenate(outs, axis=1)
    scr_ref[pl.ds(g * seq_len, seq_len), :] = attn_g

    @pl.when(g == NGROUPS - 1)
    def _():
        parts = [scr_ref[gg * seq_len:(gg + 1) * seq_len, :]
                 for gg in range(NGROUPS - 1)]
        full = jnp.concatenate(parts + [attn_g], axis=1)
        o_ref[:] = jax.lax.dot_general(
            full.astype(jnp.bfloat16), wo_ref[:].astype(jnp.bfloat16), dn,
            preferred_element_type=jnp.float32) + bo_ref[:]


def kernel(x, Wq, bq, Wk, bk, Wv, bv, Wo, bo):
    Bc, S, D = x.shape
    x2 = x.reshape(S, D)
    gw = HPG * HEAD_DIM

    wspec = pl.BlockSpec((gw, D), lambda g: (g, 0))
    bspec = pl.BlockSpec((1, gw), lambda g: (0, g))
    cspec = pl.BlockSpec((S, D), lambda g: (0, 0))
    wospec = pl.BlockSpec((D, D), lambda g: (0, 0))
    c1spec = pl.BlockSpec((1, D), lambda g: (0, 0))

    y = pl.pallas_call(
        functools.partial(_fused_body, seq_len=S),
        grid=(NGROUPS,),
        in_specs=[cspec, wspec, bspec, wspec, bspec, wspec, bspec,
                  wospec, c1spec],
        out_specs=cspec,
        out_shape=jax.ShapeDtypeStruct((S, D), jnp.float32),
        scratch_shapes=[pltpu.VMEM((NGROUPS * S, gw), jnp.float32)],
    )(x2, Wq, bq.reshape(1, D), Wk, bk.reshape(1, D),
      Wv, bv.reshape(1, D), Wo, bo.reshape(1, D))
    return y.reshape(Bc, S, D)
